# Initial kernel scaffold; baseline (speedup 1.0000x reference)
import jax
import jax.numpy as jnp
from jax.experimental import pallas as pl


def kernel(c, A, b, constraints, l, u, edge_index, phi, params):
    raise NotImplementedError("write your pallas kernel here")


# speedup vs baseline: 1.1758x; 1.1758x over previous
import jax
import jax.numpy as jnp
from jax.experimental import pallas as pl

_F32 = jnp.float32


def _dot(a, b):
    return jnp.dot(a, b, preferred_element_type=_F32)


def _dot_t(a, b):
    return jax.lax.dot_general(
        a, b, (((0,), (0,)), ((), ())), preferred_element_type=_F32)


def _mlp2(x, W1, b1, W2, b2):
    return _dot(jnp.maximum(_dot(x, W1) + b1, 0.0), W2) + b2


def _lpgcn_body(A_ref, hv0_ref, hw0_ref, *refs):
    out_ref = refs[-1]
    wrefs = refs[:-1]

    def w(i):
        return wrefs[i][...]

    E = A_ref[0]
    hv = hv0_ref[0]
    hw = hw0_ref[0]

    hv = _mlp2(hv, w(0), w(1), w(2), w(3))
    hw = _mlp2(hw, w(4), w(5), w(6), w(7))

    k = 8
    for lyr in range(4):
        Wr, Wm, bh, Wo, bo = (w(k + j) for j in range(5))
        Wr2, Wm2, bh2, Wo2, bo2 = (w(k + 20 + j) for j in range(5))
        k += 5
        q = _dot(hw, Wm)
        p = _dot(hv, Wm2)
        mv = _dot(E, q)
        mw = _dot_t(E, p)
        hv_new = _dot(jnp.maximum(_dot(hv, Wr) + mv + bh, 0.0), Wo) + bo
        hw_new = _dot(jnp.maximum(_dot(hw, Wr2) + mw + bh2, 0.0), Wo2) + bo2
        hv, hw = hv_new, hw_new

    pooled = jnp.concatenate(
        [jnp.sum(hv, axis=0, keepdims=True),
         jnp.sum(hw, axis=0, keepdims=True)], axis=1)
    res = _mlp2(pooled, w(48), w(49), w(50), w(51))
    out_ref[...] = jnp.broadcast_to(res.reshape(1, 1, 1), out_ref.shape)


def kernel(c, A, b, constraints, l, u, edge_index, phi, params):
    B, m, n = A.shape
    hv0 = jnp.stack([b, constraints], axis=-1)
    hw0 = jnp.stack([c, l, u], axis=-1)

    def prep(seq):
        out = []
        for a in seq:
            a = jnp.asarray(a, _F32)
            out.append(a.reshape(1, -1) if a.ndim == 1 else a)
        return out

    wl = prep(params['fv_in']) + prep(params['fw_in'])
    for lyr in range(4):
        wl += prep(params['cv'][lyr])
    for lyr in range(4):
        wl += prep(params['cw'][lyr])
    wl += prep(params['f_out'])

    batch3 = lambda shape: pl.BlockSpec((1,) + shape[1:], lambda i: (i, 0, 0))
    wspec = lambda a: pl.BlockSpec(a.shape, lambda i: (0, 0))

    out = pl.pallas_call(
        _lpgcn_body,
        grid=(B,),
        in_specs=[batch3(A.shape), batch3(hv0.shape), batch3(hw0.shape)]
                 + [wspec(a) for a in wl],
        out_specs=pl.BlockSpec((1, 1, 128), lambda i: (i, 0, 0)),
        out_shape=jax.ShapeDtypeStruct((B, 1, 128), _F32),
    )(A, hv0, hw0, *wl)
    return out[:, 0, :1]
